# baseline (speedup 1.0000x reference)
import functools
import math

import jax
import jax.numpy as jnp
from jax import lax
from jax.experimental import pallas as pl
from jax.experimental.pallas import tpu as pltpu
from jax.experimental.pallas import tpu_sc as plsc

_B, _L, _D, _FFN = 4, 4096, 45, 128
_DP = 48
_DG = 48
_U = 45
_NC, _NS = 2, 16
_NW = _NC * _NS
_RPW = _B * _L // _NW
_SEG = _L // _RPW
_G = 8
_GL = _L // _G


def _sc_gather(table, idx):
    mesh = plsc.VectorSubcoreMesh(core_axis_name="c", subcore_axis_name="s",
                                  num_cores=_NC, num_subcores=_NS)

    @functools.partial(
        pl.kernel,
        out_type=jax.ShapeDtypeStruct((_B * _L, _DG), jnp.float32),
        mesh=mesh,
        scratch_types=[
            pltpu.VMEM((_RPW,), jnp.int32),
            pltpu.VMEM((_RPW, _DG), jnp.float32),
            pltpu.SemaphoreType.DMA,
        ],
        compiler_params=pltpu.CompilerParams(use_tc_tiling_on_sc=False),
    )
    def gather_kernel(table_hbm, idx_hbm, out_hbm, idx_v, rows_v, sem):
        wid = lax.axis_index("s") * _NC + lax.axis_index("c")
        b = wid // _SEG
        seg = wid % _SEG
        pltpu.sync_copy(idx_hbm.at[pl.ds(seg * _RPW, _RPW)], idx_v)
        off = b * _L

        def add_off(i, carry):
            sl = pl.ds(i * 16, 16)
            idx_v[sl] = idx_v[sl] + off
            return carry

        lax.fori_loop(0, _RPW // 16, add_off, 0)
        pltpu.async_copy(table_hbm.at[idx_v], rows_v, sem).wait()
        pltpu.sync_copy(rows_v, out_hbm.at[pl.ds(wid * _RPW, _RPW)])

    return gather_kernel(table, idx)


def _stage_b(x48, ks48):
    QB, KB = 1024, 512

    def body(q_ref, k_ref, m_ref):
        for qb in range(_L // QB):
            q = q_ref[0, pl.ds(qb * QB, QB), :]
            mx = jnp.full((QB,), -jnp.inf, jnp.float32)
            sm = jnp.zeros((QB,), jnp.float32)
            for kb in range(_L // KB):
                k = k_ref[0, pl.ds(kb * KB, KB), :]
                s = lax.dot_general(q, k, (((1,), (1,)), ((), ())),
                                    preferred_element_type=jnp.float32)
                mx = jnp.maximum(mx, jnp.max(s, axis=1))
                sm = sm + jnp.sum(s, axis=1)
            m_ref[0, 0, pl.ds(qb * QB, QB)] = mx - sm * (1.0 / _L)

    return pl.pallas_call(
        body,
        grid=(_B,),
        in_specs=[pl.BlockSpec((1, _L, _DP), lambda b: (b, 0, 0)),
                  pl.BlockSpec((1, _L, _DP), lambda b: (b, 0, 0))],
        out_specs=pl.BlockSpec((1, 1, _L), lambda b: (b, 0, 0)),
        out_shape=jax.ShapeDtypeStruct((_B, 1, _L), jnp.float32),
    )(x48, ks48)


def _layer_norm_rows(v, g, b, eps=1e-12):
    mean = jnp.mean(v, axis=1, keepdims=True)
    var = jnp.mean((v - mean) ** 2, axis=1, keepdims=True)
    return g * (v - mean) / jnp.sqrt(var + eps) + b


def _stage_c(x, m2, gamma1, beta1, gamma2, beta2, w1, b1, w2, b2):
    scale = 1.0 / math.sqrt(_D)

    def body(x_ref, m_ref, g1_ref, bt1_ref, g2_ref, bt2_ref,
             w1_ref, b1_ref, w2_ref, b2_ref, o_ref, msc, vals, idxs, oh):
        msc[...] = m_ref[...].reshape(_B, _G, _GL)
        lane_g = lax.broadcasted_iota(jnp.int32, (_B, _G, _GL), 2)
        goff = lax.broadcasted_iota(jnp.int32, (_B, _G), 1) * _GL
        lane1 = lax.broadcasted_iota(jnp.int32, (1, _L), 1)

        def p1(r, carry):
            m = msc[...]
            gm = jnp.max(m, axis=2, keepdims=True)
            gi = jnp.min(jnp.where(m == gm, lane_g, _L), axis=2)
            msc[...] = jnp.where(lane_g == gi[:, :, None], -jnp.inf, m)
            gflat = jnp.reshape(gm, (_B, _G))
            gidx = gi + goff
            for b in range(_B):
                vals[b, r] = lax.slice(gflat, (b, 0), (b + 1, _G))
                idxs[b, r] = lax.slice(gidx, (b, 0), (b + 1, _G))
            return carry

        lax.fori_loop(0, _U, p1, 0)

        def p2(r, carry):
            for b in range(_B):
                vb = vals[b].reshape(_U, _G)
                ib = idxs[b].reshape(_U, _G)
                gmax = jnp.max(vb)
                sel = vb == gmax
                cidx = jnp.min(jnp.where(sel, ib, _L))
                hit = jnp.logical_and(sel, ib == cidx)
                vals[b] = jnp.where(hit, -jnp.inf, vb).reshape(_U, 1, _G)
                oh[b, r] = jnp.where(lane1 == cidx, 1.0, 0.0)
            return carry

        lax.fori_loop(0, _U, p2, 0)

        for b in range(_B):
            xb = x_ref[b]
            ohb = oh[b].reshape(_U, _L)
            qr = lax.dot_general(ohb, xb, (((1,), (0,)), ((), ())),
                                 preferred_element_type=jnp.float32)
            s2 = lax.dot_general(qr, xb, (((1,), (1,)), ((), ())),
                                 preferred_element_type=jnp.float32) * scale
            p = jnp.exp(s2 - jnp.max(s2, axis=1, keepdims=True))
            p = p / jnp.sum(p, axis=1, keepdims=True)
            attn = lax.dot_general(p, xb, (((1,), (0,)), ((), ())),
                                   preferred_element_type=jnp.float32)
            h = _layer_norm_rows(attn, g1_ref[...], bt1_ref[...])
            a = lax.dot_general(h, w1_ref[...], (((0,), (0,)), ((), ())),
                                preferred_element_type=jnp.float32)
            g = jnp.maximum(a + b1_ref[...], 0.0)
            h2 = lax.dot_general(w2_ref[...], g, (((0,), (1,)), ((), ())),
                                 preferred_element_type=jnp.float32)
            h2 = h2 + jnp.reshape(b2_ref[...], (_D, 1))
            o_ref[b] = _layer_norm_rows(h2, g2_ref[...], bt2_ref[...])

    return pl.pallas_call(
        body,
        out_shape=jax.ShapeDtypeStruct((_B, _U, _D), jnp.float32),
        scratch_shapes=[pltpu.VMEM((_B, _G, _GL), jnp.float32),
                        pltpu.VMEM((_B, _U, 1, _G), jnp.float32),
                        pltpu.VMEM((_B, _U, 1, _G), jnp.int32),
                        pltpu.VMEM((_B, _U, 1, _L), jnp.float32)],
    )(x, m2, gamma1, beta1, gamma2, beta2, w1, b1, w2, b2)


def kernel(x, gamma1, beta1, gamma2, beta2, W1, b1, W2, b2, index_sample):
    xg = jnp.pad(x, ((0, 0), (0, 0), (0, _DP - _D)))
    table = xg.reshape(_B * _L, _DG)
    ks = _sc_gather(table, index_sample.astype(jnp.int32))
    ks48 = ks.reshape(_B, _L, _DP)
    x48 = xg
    m3 = _stage_b(x48, ks48)
    m2 = m3.reshape(_B, _L)
    return _stage_c(x, m2, gamma1, beta1, gamma2, beta2, W1, b1, W2, b2)


# speedup vs baseline: 1.5986x; 1.5986x over previous
import functools
import math

import jax
import jax.numpy as jnp
from jax import lax
from jax.experimental import pallas as pl
from jax.experimental.pallas import tpu as pltpu
from jax.experimental.pallas import tpu_sc as plsc

_B, _L, _D, _FFN = 4, 4096, 45, 128
_DP = 48
_U = 45
_NC, _NS = 2, 16
_CW = 16
_IPT = _L // _NS


def _sc_counts(idx):
    mesh = plsc.VectorSubcoreMesh(core_axis_name="c", subcore_axis_name="s",
                                  num_cores=_NC, num_subcores=_NS)

    @functools.partial(
        pl.kernel,
        out_type=jax.ShapeDtypeStruct((_L, _CW), jnp.float32),
        mesh=mesh,
        scratch_types=[
            pltpu.VMEM((_IPT,), jnp.int32),
            pltpu.VMEM((_IPT, _CW), jnp.float32),
            pltpu.VMEM_SHARED((_L, _CW), jnp.float32),
        ],
        compiler_params=pltpu.CompilerParams(use_tc_tiling_on_sc=False),
    )
    def counts_kernel(idx_hbm, out_hbm, idx_v, buf_v, shared):
        c = lax.axis_index("c")
        t = lax.axis_index("s")

        @pl.when(c == 0)
        def _():
            def fill_zero(i, carry):
                buf_v[i, :] = jnp.zeros((_CW,), jnp.float32)
                return carry

            lax.fori_loop(0, _IPT, fill_zero, 0)
            pltpu.sync_copy(buf_v, shared.at[pl.ds(t * _IPT, _IPT)])
            pltpu.sync_copy(idx_hbm.at[pl.ds(t * _IPT, _IPT)], idx_v)

            def fill_one(i, carry):
                buf_v[i, :] = jnp.ones((_CW,), jnp.float32)
                return carry

            lax.fori_loop(0, _IPT, fill_one, 0)
            plsc.subcore_barrier()
            pltpu.sync_copy(buf_v, shared.at[idx_v], add=True)
            plsc.subcore_barrier()
            pltpu.sync_copy(shared.at[pl.ds(t * _IPT, _IPT)],
                            out_hbm.at[pl.ds(t * _IPT, _IPT)])

    return counts_kernel(idx)


def _stage_b(x48, cnt):
    QB, KB = 1024, 512

    def body(x_ref, c_ref, m_ref):
        crow = c_ref[0]
        s_vec = lax.dot_general(crow, x_ref[0], (((1,), (0,)), ((), ())),
                                preferred_element_type=jnp.float32)
        for qb in range(_L // QB):
            q = x_ref[0, pl.ds(qb * QB, QB), :]
            qs = lax.dot_general(q, s_vec, (((1,), (1,)), ((), ())),
                                 preferred_element_type=jnp.float32)
            mx = jnp.full((QB,), -jnp.inf, jnp.float32)
            for kb in range(_L // KB):
                k = x_ref[0, pl.ds(kb * KB, KB), :]
                cb = c_ref[0, 0, pl.ds(kb * KB, KB)]
                bias = jnp.reshape(jnp.where(cb > 0.0, 0.0, -1e30), (1, KB))
                s = lax.dot_general(q, k, (((1,), (1,)), ((), ())),
                                    preferred_element_type=jnp.float32)
                mx = jnp.maximum(mx, jnp.max(s + bias, axis=1))
            m_ref[0, 0, pl.ds(qb * QB, QB)] = mx - jnp.reshape(qs, (QB,)) * (1.0 / _L)

    return pl.pallas_call(
        body,
        grid=(_B,),
        in_specs=[pl.BlockSpec((1, _L, _DP), lambda b: (b, 0, 0)),
                  pl.BlockSpec((1, 1, _L), lambda b: (0, 0, 0))],
        out_specs=pl.BlockSpec((1, 1, _L), lambda b: (b, 0, 0)),
        out_shape=jax.ShapeDtypeStruct((_B, 1, _L), jnp.float32),
    )(x48, cnt)


def _layer_norm_rows(v, g, b, eps=1e-12):
    mean = jnp.mean(v, axis=1, keepdims=True)
    var = jnp.mean((v - mean) ** 2, axis=1, keepdims=True)
    return g * (v - mean) / jnp.sqrt(var + eps) + b


def _stage_c(x, m2, gamma1, beta1, gamma2, beta2, w1, b1, w2, b2):
    scale = 1.0 / math.sqrt(_D)

    def body(x_ref, m_ref, g1_ref, bt1_ref, g2_ref, bt2_ref,
             w1_ref, b1_ref, w2_ref, b2_ref, o_ref, msc, oh):
        msc[...] = m_ref[...]
        lane = lax.broadcasted_iota(jnp.int32, (_B, _L), 1)
        lane1 = lax.broadcasted_iota(jnp.int32, (1, _L), 1)

        def step(r, carry):
            m = msc[...]
            mx = jnp.max(m, axis=1, keepdims=True)
            idx = jnp.min(jnp.where(m == mx, lane, _L), axis=1, keepdims=True)
            msc[...] = jnp.where(lane == idx, -jnp.inf, m)
            for b in range(_B):
                idx_b = lax.slice(idx, (b, 0), (b + 1, 1))
                oh[b, r] = jnp.where(lane1 == idx_b, 1.0, 0.0)
            return carry

        lax.fori_loop(0, _U, step, 0)

        for b in range(_B):
            xb = x_ref[b]
            ohb = oh[b].reshape(_U, _L)
            qr = lax.dot_general(ohb, xb, (((1,), (0,)), ((), ())),
                                 preferred_element_type=jnp.float32)
            s2 = lax.dot_general(qr, xb, (((1,), (1,)), ((), ())),
                                 preferred_element_type=jnp.float32) * scale
            p = jnp.exp(s2 - jnp.max(s2, axis=1, keepdims=True))
            p = p / jnp.sum(p, axis=1, keepdims=True)
            attn = lax.dot_general(p, xb, (((1,), (0,)), ((), ())),
                                   preferred_element_type=jnp.float32)
            h = _layer_norm_rows(attn, g1_ref[...], bt1_ref[...])
            a = lax.dot_general(h, w1_ref[...], (((0,), (0,)), ((), ())),
                                preferred_element_type=jnp.float32)
            g = jnp.maximum(a + b1_ref[...], 0.0)
            h2 = lax.dot_general(w2_ref[...], g, (((0,), (1,)), ((), ())),
                                 preferred_element_type=jnp.float32)
            h2 = h2 + jnp.reshape(b2_ref[...], (_D, 1))
            o_ref[b] = _layer_norm_rows(h2, g2_ref[...], bt2_ref[...])

    return pl.pallas_call(
        body,
        out_shape=jax.ShapeDtypeStruct((_B, _U, _D), jnp.float32),
        scratch_shapes=[pltpu.VMEM((_B, _L), jnp.float32),
                        pltpu.VMEM((_B, _U, 1, _L), jnp.float32)],
    )(x, m2, gamma1, beta1, gamma2, beta2, w1, b1, w2, b2)


def kernel(x, gamma1, beta1, gamma2, beta2, W1, b1, W2, b2, index_sample):
    cnt16 = _sc_counts(index_sample.astype(jnp.int32))
    cnt = jnp.reshape(cnt16[:, 0], (1, 1, _L))
    x48 = jnp.pad(x, ((0, 0), (0, 0), (0, _DP - _D)))
    m3 = _stage_b(x48, cnt)
    m2 = m3.reshape(_B, _L)
    return _stage_c(x, m2, gamma1, beta1, gamma2, beta2, W1, b1, W2, b2)
